# baseline (speedup 1.0000x reference)
import functools

import jax
import jax.numpy as jnp
from jax import lax
from jax.experimental import pallas as pl
from jax.experimental.pallas import tpu as pltpu
from jax.experimental.pallas import tpu_sc as plsc

D_MODEL = 64
BATCH = 1024
SEQ = 200
MAX_LENGTH = 1024
VOCAB = 1000000

NUM_WORKERS = 32
CHUNK = 128
N_CHUNKS = BATCH * SEQ // CHUNK
CHUNKS_PER_W = N_CHUNKS // NUM_WORKERS
NBUF = 2
N_T = CHUNKS_PER_W // NBUF
LANES = 16
SLICES = D_MODEL // LANES
HSEQ = SEQ // 2

PACK_COLS = 512
PACK_GRID = (VOCAB + 2 * PACK_COLS - 1) // (2 * PACK_COLS)


def _positional_encoding(length, depth):
    depth = depth / 2
    positions = jnp.arange(length, dtype=jnp.float32)[:, None]
    depths = jnp.arange(depth, dtype=jnp.float32)[None, :] / depth
    angle_rates = 1.0 / jnp.power(10000.0, depths)
    angle_rads = positions * angle_rates
    pos = jnp.concatenate([jnp.sin(angle_rads), jnp.cos(angle_rads)], axis=-1)
    return pos.astype(jnp.float32)


def _pack_kernel(lo_ref, hi_ref, out_ref):
    out_ref[:, 0:D_MODEL] = lo_ref[...].T
    out_ref[:, D_MODEL:2 * D_MODEL] = hi_ref[...].T


_pack = pl.pallas_call(
    _pack_kernel,
    grid=(PACK_GRID,),
    in_specs=[
        pl.BlockSpec((D_MODEL, PACK_COLS), lambda i: (0, 2 * i)),
        pl.BlockSpec((D_MODEL, PACK_COLS), lambda i: (0, 2 * i + 1)),
    ],
    out_specs=pl.BlockSpec((PACK_COLS, 2 * D_MODEL), lambda i: (i, 0)),
    out_shape=jax.ShapeDtypeStruct((PACK_GRID * PACK_COLS, 2 * D_MODEL),
                                   jnp.float32),
)


_MESH = plsc.VectorSubcoreMesh(core_axis_name="c", subcore_axis_name="s")


@functools.partial(
    pl.kernel,
    mesh=_MESH,
    compiler_params=pltpu.CompilerParams(use_tc_tiling_on_sc=True),
    out_type=jax.ShapeDtypeStruct((BATCH * SEQ, D_MODEL), jnp.float32),
    scratch_types=[
        pltpu.VMEM((CHUNKS_PER_W, 1, CHUNK), jnp.int32),
        pltpu.VMEM((HSEQ, 2 * D_MODEL), jnp.float32),
        pltpu.VMEM((NBUF, CHUNK, 2 * D_MODEL), jnp.float32),
        pltpu.VMEM((NBUF, CHUNK, D_MODEL), jnp.float32),
        pltpu.VMEM((NBUF, CHUNK, LANES), jnp.int32),
        pltpu.SemaphoreType.DMA,
        pltpu.SemaphoreType.DMA,
        pltpu.SemaphoreType.DMA,
    ],
)
def _emb_kernel(xpair_hbm, xpar_hbm, pos_hbm, tpack_hbm, out_hbm,
                idx_v, pos_v, rows_v, res_v, par_v, gsem, psem, osem):
    wid = lax.axis_index("s") * 2 + lax.axis_index("c")
    chunk_base = wid * CHUNKS_PER_W
    pltpu.sync_copy(xpair_hbm.at[pl.ds(chunk_base, CHUNKS_PER_W)], idx_v)
    pltpu.sync_copy(pos_hbm, pos_v)

    def start_gather(c, slot):
        pltpu.async_copy(tpack_hbm.at[idx_v.at[c, 0]], rows_v.at[slot], gsem)
        pltpu.async_copy(xpar_hbm.at[chunk_base + c], par_v.at[slot], psem)

    def wait_gather(slot):
        pltpu.make_async_copy(tpack_hbm.at[pl.ds(0, CHUNK)],
                              rows_v.at[slot], gsem).wait()
        pltpu.make_async_copy(xpar_hbm.at[0], par_v.at[slot], psem).wait()

    def start_out(c, slot):
        base = pl.multiple_of((chunk_base + c) * CHUNK, CHUNK)
        pltpu.async_copy(res_v.at[slot], out_hbm.at[pl.ds(base, CHUNK)], osem)

    def wait_out(slot):
        pltpu.make_async_copy(res_v.at[slot],
                              out_hbm.at[pl.ds(0, CHUNK)], osem).wait()

    start_gather(0, 0)

    def t_body(t, carry):
        for b in range(NBUF):
            c = NBUF * t + b
            nslot = (b + 1) % NBUF
            if b == NBUF - 1:
                @pl.when(t < N_T - 1)
                def _():
                    wait_out(nslot)
                    start_gather(c + 1, nslot)
            else:
                @pl.when(t >= 1)
                def _():
                    wait_out(nslot)
                start_gather(c + 1, nslot)
            wait_gather(b)
            s0 = lax.rem((chunk_base + c) * CHUNK, SEQ)

            def r_body(r, carry2):
                m = par_v[b, r, :] > 0
                sr = s0 + r
                sr = sr - lax.select(sr >= SEQ, SEQ, 0)
                prow = sr - lax.select(sr >= HSEQ, HSEQ, 0)
                pcol = lax.select(sr >= HSEQ, D_MODEL, 0)
                for g in range(SLICES):
                    lo = rows_v[b, r, pl.ds(g * LANES, LANES)]
                    hi = rows_v[b, r, pl.ds(D_MODEL + g * LANES, LANES)]
                    val = jnp.where(m, hi, lo)
                    res_v[b, r, pl.ds(g * LANES, LANES)] = (
                        val * 8.0 + pos_v[prow, pl.ds(pcol + g * LANES, LANES)])
                return carry2

            lax.fori_loop(0, CHUNK, r_body, 0)
            start_out(c, b)
        return carry

    lax.fori_loop(0, N_T, t_body, 0)
    for b in range(NBUF):
        wait_out(b)


def kernel(x, table):
    pos = _positional_encoding(MAX_LENGTH, D_MODEL)[:SEQ]
    pos2 = jnp.concatenate([pos[:HSEQ], pos[HSEQ:]], axis=1)
    tpack = _pack(table.T, table.T)
    xi = x.astype(jnp.int32)
    xpair = ((xi >> 10) * PACK_COLS + (xi & (PACK_COLS - 1))).reshape(
        N_CHUNKS, 1, CHUNK)
    xpar = jnp.broadcast_to(
        ((xi >> 9) & 1).reshape(N_CHUNKS, CHUNK, 1), (N_CHUNKS, CHUNK, LANES)
    ).astype(jnp.int32)
    out = _emb_kernel(xpair, xpar, pos2, tpack)
    return out.reshape(BATCH, SEQ, D_MODEL)


# speedup vs baseline: 1.7917x; 1.7917x over previous
import functools

import jax
import jax.numpy as jnp
from jax import lax
from jax.experimental import pallas as pl
from jax.experimental.pallas import tpu as pltpu
from jax.experimental.pallas import tpu_sc as plsc

D_MODEL = 64
BATCH = 1024
SEQ = 200
MAX_LENGTH = 1024
VOCAB = 1000000

NUM_WORKERS = 32
CHUNK = 128
N_CHUNKS = BATCH * SEQ // CHUNK
CHUNKS_PER_W = N_CHUNKS // NUM_WORKERS
NBUF = 2
N_T = CHUNKS_PER_W // NBUF
LANES = 16
SLICES = D_MODEL // LANES
HSEQ = SEQ // 2

PACK_BLOCK = 8192
PACK_HALF = PACK_BLOCK // 2
PACK_SHIFT = 12
PACK_GRID = (VOCAB + PACK_BLOCK - 1) // PACK_BLOCK


def _positional_encoding(length, depth):
    depth = depth / 2
    positions = jnp.arange(length, dtype=jnp.float32)[:, None]
    depths = jnp.arange(depth, dtype=jnp.float32)[None, :] / depth
    angle_rates = 1.0 / jnp.power(10000.0, depths)
    angle_rads = positions * angle_rates
    pos = jnp.concatenate([jnp.sin(angle_rads), jnp.cos(angle_rads)], axis=-1)
    return pos.astype(jnp.float32)


def _pack_kernel(tt_ref, out_ref):
    blk = tt_ref[...]
    out_ref[:, 0:D_MODEL] = blk[:, 0:PACK_HALF].T
    out_ref[:, D_MODEL:2 * D_MODEL] = blk[:, PACK_HALF:PACK_BLOCK].T


_pack = pl.pallas_call(
    _pack_kernel,
    grid=(PACK_GRID,),
    in_specs=[pl.BlockSpec((D_MODEL, PACK_BLOCK), lambda i: (0, i))],
    out_specs=pl.BlockSpec((PACK_HALF, 2 * D_MODEL), lambda i: (i, 0)),
    out_shape=jax.ShapeDtypeStruct((PACK_GRID * PACK_HALF, 2 * D_MODEL),
                                   jnp.float32),
)


_MESH = plsc.VectorSubcoreMesh(core_axis_name="c", subcore_axis_name="s")


@functools.partial(
    pl.kernel,
    mesh=_MESH,
    compiler_params=pltpu.CompilerParams(use_tc_tiling_on_sc=True),
    out_type=jax.ShapeDtypeStruct((BATCH * SEQ, D_MODEL), jnp.float32),
    scratch_types=[
        pltpu.VMEM((CHUNKS_PER_W, 1, CHUNK), jnp.int32),
        pltpu.VMEM((HSEQ, 2 * D_MODEL), jnp.float32),
        pltpu.VMEM((NBUF, CHUNK, 2 * D_MODEL), jnp.float32),
        pltpu.VMEM((NBUF, CHUNK, D_MODEL), jnp.float32),
        pltpu.VMEM((NBUF, CHUNK, LANES), jnp.int32),
        pltpu.SemaphoreType.DMA,
        pltpu.SemaphoreType.DMA,
        pltpu.SemaphoreType.DMA,
    ],
)
def _emb_kernel(xpair_hbm, xpar_hbm, pos_hbm, tpack_hbm, out_hbm,
                idx_v, pos_v, rows_v, res_v, par_v, gsem, psem, osem):
    wid = lax.axis_index("s") * 2 + lax.axis_index("c")
    chunk_base = wid * CHUNKS_PER_W
    pltpu.sync_copy(xpair_hbm.at[pl.ds(chunk_base, CHUNKS_PER_W)], idx_v)
    pltpu.sync_copy(pos_hbm, pos_v)

    def start_gather(c, slot):
        pltpu.async_copy(tpack_hbm.at[idx_v.at[c, 0]], rows_v.at[slot], gsem)
        pltpu.async_copy(xpar_hbm.at[chunk_base + c], par_v.at[slot], psem)

    def wait_gather(slot):
        pltpu.make_async_copy(tpack_hbm.at[pl.ds(0, CHUNK)],
                              rows_v.at[slot], gsem).wait()
        pltpu.make_async_copy(xpar_hbm.at[0], par_v.at[slot], psem).wait()

    def start_out(c, slot):
        base = pl.multiple_of((chunk_base + c) * CHUNK, CHUNK)
        pltpu.async_copy(res_v.at[slot], out_hbm.at[pl.ds(base, CHUNK)], osem)

    def wait_out(slot):
        pltpu.make_async_copy(res_v.at[slot],
                              out_hbm.at[pl.ds(0, CHUNK)], osem).wait()

    start_gather(0, 0)

    def t_body(t, carry):
        for b in range(NBUF):
            c = NBUF * t + b
            nslot = (b + 1) % NBUF
            if b == NBUF - 1:
                @pl.when(t < N_T - 1)
                def _():
                    wait_out(nslot)
                    start_gather(c + 1, nslot)
            else:
                @pl.when(t >= 1)
                def _():
                    wait_out(nslot)
                start_gather(c + 1, nslot)
            wait_gather(b)
            s0 = lax.rem((chunk_base + c) * CHUNK, SEQ)

            def r_body(r, carry2):
                m = par_v[b, r, :] > 0
                sr = s0 + r
                sr = sr - lax.select(sr >= SEQ, SEQ, 0)
                prow = sr - lax.select(sr >= HSEQ, HSEQ, 0)
                pcol = lax.select(sr >= HSEQ, D_MODEL, 0)
                for g in range(SLICES):
                    lo = rows_v[b, r, pl.ds(g * LANES, LANES)]
                    hi = rows_v[b, r, pl.ds(D_MODEL + g * LANES, LANES)]
                    val = jnp.where(m, hi, lo)
                    res_v[b, r, pl.ds(g * LANES, LANES)] = (
                        val * 8.0 + pos_v[prow, pl.ds(pcol + g * LANES, LANES)])
                return carry2

            lax.fori_loop(0, CHUNK, r_body, 0)
            start_out(c, b)
        return carry

    lax.fori_loop(0, N_T, t_body, 0)
    for b in range(NBUF):
        wait_out(b)


def kernel(x, table):
    pos = _positional_encoding(MAX_LENGTH, D_MODEL)[:SEQ]
    pos2 = jnp.concatenate([pos[:HSEQ], pos[HSEQ:]], axis=1)
    tpack = _pack(table.T)
    xi = x.astype(jnp.int32)
    xpair = ((xi >> (PACK_SHIFT + 1)) * PACK_HALF
             + (xi & (PACK_HALF - 1))).reshape(N_CHUNKS, 1, CHUNK)
    xpar = jnp.broadcast_to(
        ((xi >> PACK_SHIFT) & 1).reshape(N_CHUNKS, CHUNK, 1),
        (N_CHUNKS, CHUNK, LANES)).astype(jnp.int32)
    out = _emb_kernel(xpair, xpar, pos2, tpack)
    return out.reshape(BATCH, SEQ, D_MODEL)


# speedup vs baseline: 1.7980x; 1.0035x over previous
import functools

import jax
import jax.numpy as jnp
from jax import lax
from jax.experimental import pallas as pl
from jax.experimental.pallas import tpu as pltpu
from jax.experimental.pallas import tpu_sc as plsc

D_MODEL = 64
BATCH = 1024
SEQ = 200
MAX_LENGTH = 1024
VOCAB = 1000000

NUM_WORKERS = 32
CHUNK = 128
N_CHUNKS = BATCH * SEQ // CHUNK
CHUNKS_PER_W = N_CHUNKS // NUM_WORKERS
NBUF = 2
N_T = CHUNKS_PER_W // NBUF
LANES = 16
SLICES = D_MODEL // LANES
HSEQ = SEQ // 2

PACK_BLOCK = 8192
PACK_HALF = PACK_BLOCK // 2
PACK_SHIFT = 12
PACK_GRID = (VOCAB + PACK_BLOCK - 1) // PACK_BLOCK


def _positional_encoding(length, depth):
    depth = depth / 2
    positions = jnp.arange(length, dtype=jnp.float32)[:, None]
    depths = jnp.arange(depth, dtype=jnp.float32)[None, :] / depth
    angle_rates = 1.0 / jnp.power(10000.0, depths)
    angle_rads = positions * angle_rates
    pos = jnp.concatenate([jnp.sin(angle_rads), jnp.cos(angle_rads)], axis=-1)
    return pos.astype(jnp.float32)


def _pack_kernel(tt_ref, out_ref):
    blk = tt_ref[...]
    out_ref[:, 0:D_MODEL] = blk[:, 0:PACK_HALF].T
    out_ref[:, D_MODEL:2 * D_MODEL] = blk[:, PACK_HALF:PACK_BLOCK].T


_pack = pl.pallas_call(
    _pack_kernel,
    grid=(PACK_GRID,),
    in_specs=[pl.BlockSpec((D_MODEL, PACK_BLOCK), lambda i: (0, i))],
    out_specs=pl.BlockSpec((PACK_HALF, 2 * D_MODEL), lambda i: (i, 0)),
    out_shape=jax.ShapeDtypeStruct((PACK_GRID * PACK_HALF, 2 * D_MODEL),
                                   jnp.float32),
)


_MESH = plsc.VectorSubcoreMesh(core_axis_name="c", subcore_axis_name="s")


@functools.partial(
    pl.kernel,
    mesh=_MESH,
    compiler_params=pltpu.CompilerParams(use_tc_tiling_on_sc=True),
    out_type=jax.ShapeDtypeStruct((BATCH * SEQ, D_MODEL), jnp.float32),
    scratch_types=[
        pltpu.VMEM((CHUNKS_PER_W, 1, CHUNK), jnp.int32),
        pltpu.VMEM((HSEQ, 2 * D_MODEL), jnp.float32),
        pltpu.VMEM((NBUF, CHUNK, 2 * D_MODEL), jnp.float32),
        pltpu.VMEM((NBUF, CHUNK, D_MODEL), jnp.float32),
        pltpu.VMEM((NBUF, CHUNK, LANES), jnp.int32),
        pltpu.SemaphoreType.DMA,
        pltpu.SemaphoreType.DMA,
        pltpu.SemaphoreType.DMA,
    ],
)
def _emb_kernel(xpair_hbm, xpar_hbm, pos_hbm, tpack_hbm, out_hbm,
                idx_v, pos_v, rows_v, res_v, par_v, gsem, psem, osem):
    wid = lax.axis_index("s") * 2 + lax.axis_index("c")
    chunk_base = wid * CHUNKS_PER_W
    pltpu.sync_copy(xpair_hbm.at[pl.ds(chunk_base, CHUNKS_PER_W)], idx_v)
    pltpu.sync_copy(pos_hbm, pos_v)

    def start_gather(c, slot):
        pltpu.async_copy(tpack_hbm.at[idx_v.at[c, 0]], rows_v.at[slot], gsem)
        pltpu.async_copy(xpar_hbm.at[chunk_base + c], par_v.at[slot], psem)

    def wait_gather(slot):
        pltpu.make_async_copy(tpack_hbm.at[pl.ds(0, CHUNK)],
                              rows_v.at[slot], gsem).wait()
        pltpu.make_async_copy(xpar_hbm.at[0], par_v.at[slot], psem).wait()

    def start_out(c, slot):
        base = pl.multiple_of((chunk_base + c) * CHUNK, CHUNK)
        pltpu.async_copy(res_v.at[slot], out_hbm.at[pl.ds(base, CHUNK)], osem)

    def wait_out(slot):
        pltpu.make_async_copy(res_v.at[slot],
                              out_hbm.at[pl.ds(0, CHUNK)], osem).wait()

    start_gather(0, 0)

    def t_body(t, carry):
        for b in range(NBUF):
            c = NBUF * t + b
            nslot = (b + 1) % NBUF
            if b == NBUF - 1:
                @pl.when(t < N_T - 1)
                def _():
                    wait_out(nslot)
                    start_gather(c + 1, nslot)
            else:
                @pl.when(t >= 1)
                def _():
                    wait_out(nslot)
                start_gather(c + 1, nslot)
            wait_gather(b)
            s0 = lax.rem((chunk_base + c) * CHUNK, SEQ)

            def r_body(r2, carry2):
                for dr in range(2):
                    r = 2 * r2 + dr
                    m = par_v[b, r, :] > 0
                    sr = s0 + r
                    sr = sr - lax.select(sr >= SEQ, SEQ, 0)
                    prow = sr - lax.select(sr >= HSEQ, HSEQ, 0)
                    pcol = lax.select(sr >= HSEQ, D_MODEL, 0)
                    for g in range(SLICES):
                        lo = rows_v[b, r, pl.ds(g * LANES, LANES)]
                        hi = rows_v[b, r, pl.ds(D_MODEL + g * LANES, LANES)]
                        val = jnp.where(m, hi, lo)
                        res_v[b, r, pl.ds(g * LANES, LANES)] = (
                            val * 8.0
                            + pos_v[prow, pl.ds(pcol + g * LANES, LANES)])
                return carry2

            lax.fori_loop(0, CHUNK // 2, r_body, 0)
            start_out(c, b)
        return carry

    lax.fori_loop(0, N_T, t_body, 0)
    for b in range(NBUF):
        wait_out(b)


def kernel(x, table):
    pos = _positional_encoding(MAX_LENGTH, D_MODEL)[:SEQ]
    pos2 = jnp.concatenate([pos[:HSEQ], pos[HSEQ:]], axis=1)
    tpack = _pack(table.T)
    xi = x.astype(jnp.int32)
    xpair = ((xi >> (PACK_SHIFT + 1)) * PACK_HALF
             + (xi & (PACK_HALF - 1))).reshape(N_CHUNKS, 1, CHUNK)
    xpar = jnp.broadcast_to(
        ((xi >> PACK_SHIFT) & 1).reshape(N_CHUNKS, CHUNK, 1),
        (N_CHUNKS, CHUNK, LANES)).astype(jnp.int32)
    out = _emb_kernel(xpair, xpar, pos2, tpack)
    return out.reshape(BATCH, SEQ, D_MODEL)


# speedup vs baseline: 1.9077x; 1.0610x over previous
import functools

import jax
import jax.numpy as jnp
from jax import lax
from jax.experimental import pallas as pl
from jax.experimental.pallas import tpu as pltpu
from jax.experimental.pallas import tpu_sc as plsc

D_MODEL = 64
BATCH = 1024
SEQ = 200
MAX_LENGTH = 1024
VOCAB = 1000000

NUM_WORKERS = 32
CHUNK = 128
N_CHUNKS = BATCH * SEQ // CHUNK
CHUNKS_PER_W = N_CHUNKS // NUM_WORKERS
NBUF = 2
N_T = CHUNKS_PER_W // NBUF
LANES = 16
SLICES = D_MODEL // LANES
HSEQ = SEQ // 2

PACK_BLOCK = 16384
PACK_HALF = PACK_BLOCK // 2
PACK_SHIFT = 13
PACK_GRID = (VOCAB + PACK_BLOCK - 1) // PACK_BLOCK


def _positional_encoding(length, depth):
    depth = depth / 2
    positions = jnp.arange(length, dtype=jnp.float32)[:, None]
    depths = jnp.arange(depth, dtype=jnp.float32)[None, :] / depth
    angle_rates = 1.0 / jnp.power(10000.0, depths)
    angle_rads = positions * angle_rates
    pos = jnp.concatenate([jnp.sin(angle_rads), jnp.cos(angle_rads)], axis=-1)
    return pos.astype(jnp.float32)


def _pack_kernel(tt_ref, out_ref):
    blk = tt_ref[...]
    out_ref[:, 0:D_MODEL] = blk[:, 0:PACK_HALF].T
    out_ref[:, D_MODEL:2 * D_MODEL] = blk[:, PACK_HALF:PACK_BLOCK].T


_pack = pl.pallas_call(
    _pack_kernel,
    grid=(PACK_GRID,),
    in_specs=[pl.BlockSpec((D_MODEL, PACK_BLOCK), lambda i: (0, i))],
    out_specs=pl.BlockSpec((PACK_HALF, 2 * D_MODEL), lambda i: (i, 0)),
    out_shape=jax.ShapeDtypeStruct((PACK_GRID * PACK_HALF, 2 * D_MODEL),
                                   jnp.float32),
)


_MESH = plsc.VectorSubcoreMesh(core_axis_name="c", subcore_axis_name="s")


@functools.partial(
    pl.kernel,
    mesh=_MESH,
    compiler_params=pltpu.CompilerParams(use_tc_tiling_on_sc=True),
    out_type=jax.ShapeDtypeStruct((BATCH * SEQ, D_MODEL), jnp.float32),
    scratch_types=[
        pltpu.VMEM((CHUNKS_PER_W, 1, CHUNK), jnp.int32),
        pltpu.VMEM((HSEQ, 2 * D_MODEL), jnp.float32),
        pltpu.VMEM((NBUF, CHUNK, 2 * D_MODEL), jnp.float32),
        pltpu.VMEM((NBUF, CHUNK, D_MODEL), jnp.float32),
        pltpu.VMEM((NBUF, CHUNK, LANES), jnp.int32),
        pltpu.SemaphoreType.DMA,
        pltpu.SemaphoreType.DMA,
        pltpu.SemaphoreType.DMA,
    ],
)
def _emb_kernel(xpair_hbm, xpar_hbm, pos_hbm, tpack_hbm, out_hbm,
                idx_v, pos_v, rows_v, res_v, par_v, gsem, psem, osem):
    wid = lax.axis_index("s") * 2 + lax.axis_index("c")
    chunk_base = wid * CHUNKS_PER_W
    pltpu.sync_copy(xpair_hbm.at[pl.ds(chunk_base, CHUNKS_PER_W)], idx_v)
    pltpu.sync_copy(pos_hbm, pos_v)

    def start_gather(c, slot):
        pltpu.async_copy(tpack_hbm.at[idx_v.at[c, 0]], rows_v.at[slot], gsem)
        pltpu.async_copy(xpar_hbm.at[chunk_base + c], par_v.at[slot], psem)

    def wait_gather(slot):
        pltpu.make_async_copy(tpack_hbm.at[pl.ds(0, CHUNK)],
                              rows_v.at[slot], gsem).wait()
        pltpu.make_async_copy(xpar_hbm.at[0], par_v.at[slot], psem).wait()

    def start_out(c, slot):
        base = pl.multiple_of((chunk_base + c) * CHUNK, CHUNK)
        pltpu.async_copy(res_v.at[slot], out_hbm.at[pl.ds(base, CHUNK)], osem)

    def wait_out(slot):
        pltpu.make_async_copy(res_v.at[slot],
                              out_hbm.at[pl.ds(0, CHUNK)], osem).wait()

    start_gather(0, 0)

    def t_body(t, carry):
        for b in range(NBUF):
            c = NBUF * t + b
            nslot = (b + 1) % NBUF
            if b == NBUF - 1:
                @pl.when(t < N_T - 1)
                def _():
                    wait_out(nslot)
                    start_gather(c + 1, nslot)
            else:
                @pl.when(t >= 1)
                def _():
                    wait_out(nslot)
                start_gather(c + 1, nslot)
            wait_gather(b)
            s0 = lax.rem((chunk_base + c) * CHUNK, SEQ)

            def r_body(r2, carry2):
                for dr in range(2):
                    r = 2 * r2 + dr
                    m = par_v[b, r, :] > 0
                    sr = s0 + r
                    sr = sr - lax.select(sr >= SEQ, SEQ, 0)
                    prow = sr - lax.select(sr >= HSEQ, HSEQ, 0)
                    pcol = lax.select(sr >= HSEQ, D_MODEL, 0)
                    for g in range(SLICES):
                        lo = rows_v[b, r, pl.ds(g * LANES, LANES)]
                        hi = rows_v[b, r, pl.ds(D_MODEL + g * LANES, LANES)]
                        val = jnp.where(m, hi, lo)
                        res_v[b, r, pl.ds(g * LANES, LANES)] = (
                            val * 8.0
                            + pos_v[prow, pl.ds(pcol + g * LANES, LANES)])
                return carry2

            lax.fori_loop(0, CHUNK // 2, r_body, 0)
            start_out(c, b)
        return carry

    lax.fori_loop(0, N_T, t_body, 0)
    for b in range(NBUF):
        wait_out(b)


def kernel(x, table):
    pos = _positional_encoding(MAX_LENGTH, D_MODEL)[:SEQ]
    pos2 = jnp.concatenate([pos[:HSEQ], pos[HSEQ:]], axis=1)
    tpack = _pack(table.T)
    xi = x.astype(jnp.int32)
    xpair = ((xi >> (PACK_SHIFT + 1)) * PACK_HALF
             + (xi & (PACK_HALF - 1))).reshape(N_CHUNKS, 1, CHUNK)
    xpar = jnp.broadcast_to(
        ((xi >> PACK_SHIFT) & 1).reshape(N_CHUNKS, CHUNK, 1),
        (N_CHUNKS, CHUNK, LANES)).astype(jnp.int32)
    out = _emb_kernel(xpair, xpar, pos2, tpack)
    return out.reshape(BATCH, SEQ, D_MODEL)


# speedup vs baseline: 1.9637x; 1.0293x over previous
import functools

import jax
import jax.numpy as jnp
from jax import lax
from jax.experimental import pallas as pl
from jax.experimental.pallas import tpu as pltpu
from jax.experimental.pallas import tpu_sc as plsc

D_MODEL = 64
BATCH = 1024
SEQ = 200
MAX_LENGTH = 1024
VOCAB = 1000000

NUM_WORKERS = 32
CHUNK = 128
N_CHUNKS = BATCH * SEQ // CHUNK
CHUNKS_PER_W = N_CHUNKS // NUM_WORKERS
NBUF = 2
N_T = CHUNKS_PER_W // NBUF
LANES = 16
SLICES = D_MODEL // LANES
HSEQ = SEQ // 2

PACK_BLOCK = 32768
PACK_HALF = PACK_BLOCK // 2
PACK_SHIFT = 14
PACK_GRID = (VOCAB + PACK_BLOCK - 1) // PACK_BLOCK


def _positional_encoding(length, depth):
    depth = depth / 2
    positions = jnp.arange(length, dtype=jnp.float32)[:, None]
    depths = jnp.arange(depth, dtype=jnp.float32)[None, :] / depth
    angle_rates = 1.0 / jnp.power(10000.0, depths)
    angle_rads = positions * angle_rates
    pos = jnp.concatenate([jnp.sin(angle_rads), jnp.cos(angle_rads)], axis=-1)
    return pos.astype(jnp.float32)


def _pack_kernel(tt_ref, out_ref):
    blk = tt_ref[...]
    out_ref[:, 0:D_MODEL] = blk[:, 0:PACK_HALF].T
    out_ref[:, D_MODEL:2 * D_MODEL] = blk[:, PACK_HALF:PACK_BLOCK].T


_pack = pl.pallas_call(
    _pack_kernel,
    grid=(PACK_GRID,),
    in_specs=[pl.BlockSpec((D_MODEL, PACK_BLOCK), lambda i: (0, i))],
    out_specs=pl.BlockSpec((PACK_HALF, 2 * D_MODEL), lambda i: (i, 0)),
    out_shape=jax.ShapeDtypeStruct((PACK_GRID * PACK_HALF, 2 * D_MODEL),
                                   jnp.float32),
)


_MESH = plsc.VectorSubcoreMesh(core_axis_name="c", subcore_axis_name="s")


@functools.partial(
    pl.kernel,
    mesh=_MESH,
    compiler_params=pltpu.CompilerParams(use_tc_tiling_on_sc=True),
    out_type=jax.ShapeDtypeStruct((BATCH * SEQ, D_MODEL), jnp.float32),
    scratch_types=[
        pltpu.VMEM((CHUNKS_PER_W, 1, CHUNK), jnp.int32),
        pltpu.VMEM((HSEQ, 2 * D_MODEL), jnp.float32),
        pltpu.VMEM((NBUF, CHUNK, 2 * D_MODEL), jnp.float32),
        pltpu.VMEM((NBUF, CHUNK, D_MODEL), jnp.float32),
        pltpu.VMEM((NBUF, CHUNK, LANES), jnp.int32),
        pltpu.SemaphoreType.DMA,
        pltpu.SemaphoreType.DMA,
        pltpu.SemaphoreType.DMA,
    ],
)
def _emb_kernel(xpair_hbm, xpar_hbm, pos_hbm, tpack_hbm, out_hbm,
                idx_v, pos_v, rows_v, res_v, par_v, gsem, psem, osem):
    wid = lax.axis_index("s") * 2 + lax.axis_index("c")
    chunk_base = wid * CHUNKS_PER_W
    pltpu.sync_copy(xpair_hbm.at[pl.ds(chunk_base, CHUNKS_PER_W)], idx_v)
    pltpu.sync_copy(pos_hbm, pos_v)

    def start_gather(c, slot):
        pltpu.async_copy(tpack_hbm.at[idx_v.at[c, 0]], rows_v.at[slot], gsem)
        pltpu.async_copy(xpar_hbm.at[chunk_base + c], par_v.at[slot], psem)

    def wait_gather(slot):
        pltpu.make_async_copy(tpack_hbm.at[pl.ds(0, CHUNK)],
                              rows_v.at[slot], gsem).wait()
        pltpu.make_async_copy(xpar_hbm.at[0], par_v.at[slot], psem).wait()

    def start_out(c, slot):
        base = pl.multiple_of((chunk_base + c) * CHUNK, CHUNK)
        pltpu.async_copy(res_v.at[slot], out_hbm.at[pl.ds(base, CHUNK)], osem)

    def wait_out(slot):
        pltpu.make_async_copy(res_v.at[slot],
                              out_hbm.at[pl.ds(0, CHUNK)], osem).wait()

    start_gather(0, 0)

    def t_body(t, carry):
        for b in range(NBUF):
            c = NBUF * t + b
            nslot = (b + 1) % NBUF
            if b == NBUF - 1:
                @pl.when(t < N_T - 1)
                def _():
                    wait_out(nslot)
                    start_gather(c + 1, nslot)
            else:
                @pl.when(t >= 1)
                def _():
                    wait_out(nslot)
                start_gather(c + 1, nslot)
            wait_gather(b)
            s0 = lax.rem((chunk_base + c) * CHUNK, SEQ)

            def r_body(r2, carry2):
                for dr in range(2):
                    r = 2 * r2 + dr
                    m = par_v[b, r, :] > 0
                    sr = s0 + r
                    sr = sr - lax.select(sr >= SEQ, SEQ, 0)
                    prow = sr - lax.select(sr >= HSEQ, HSEQ, 0)
                    pcol = lax.select(sr >= HSEQ, D_MODEL, 0)
                    for g in range(SLICES):
                        lo = rows_v[b, r, pl.ds(g * LANES, LANES)]
                        hi = rows_v[b, r, pl.ds(D_MODEL + g * LANES, LANES)]
                        val = jnp.where(m, hi, lo)
                        res_v[b, r, pl.ds(g * LANES, LANES)] = (
                            val * 8.0
                            + pos_v[prow, pl.ds(pcol + g * LANES, LANES)])
                return carry2

            lax.fori_loop(0, CHUNK // 2, r_body, 0)
            start_out(c, b)
        return carry

    lax.fori_loop(0, N_T, t_body, 0)
    for b in range(NBUF):
        wait_out(b)


def kernel(x, table):
    pos = _positional_encoding(MAX_LENGTH, D_MODEL)[:SEQ]
    pos2 = jnp.concatenate([pos[:HSEQ], pos[HSEQ:]], axis=1)
    tpack = _pack(table.T)
    xi = x.astype(jnp.int32)
    xpair = ((xi >> (PACK_SHIFT + 1)) * PACK_HALF
             + (xi & (PACK_HALF - 1))).reshape(N_CHUNKS, 1, CHUNK)
    xpar = jnp.broadcast_to(
        ((xi >> PACK_SHIFT) & 1).reshape(N_CHUNKS, CHUNK, 1),
        (N_CHUNKS, CHUNK, LANES)).astype(jnp.int32)
    out = _emb_kernel(xpair, xpar, pos2, tpack)
    return out.reshape(BATCH, SEQ, D_MODEL)
